# Initial kernel scaffold; baseline (speedup 1.0000x reference)
import jax
import jax.numpy as jnp
from jax.experimental import pallas as pl


def kernel(h_cell, h_tissue, S, params):
    raise NotImplementedError("write your pallas kernel here")


# speedup vs baseline: 3.3815x; 3.3815x over previous
import functools
import math

import jax
import jax.numpy as jnp
from jax.experimental import pallas as pl
from jax.experimental.pallas import tpu as pltpu

D = 256
H = 4
DK = D // H
N = 4096
K = 16

NBLK_A = 8
NBLK_B = 8
BA = N // NBLK_A
BB = N // NBLK_B

_PREC = jax.lax.Precision.HIGHEST


def _lin(x, w, b=None):
    out = jax.lax.dot_general(x, w, (((1,), (1,)), ((), ())), precision=_PREC)
    if b is not None:
        out = out + b
    return out


def _layer_norm(x, g, b, eps=1e-5):
    mu = jnp.mean(x, axis=-1, keepdims=True)
    xc = x - mu
    var = jnp.mean(xc * xc, axis=-1, keepdims=True)
    return xc * jax.lax.rsqrt(var + eps) * g + b


def _bu_kernel(
    h_cell_ref,
    st_ref,
    h_tissue_ref,
    wbuq_ref, bbuq_ref, wbuk_ref, bbuk_ref, wbuv_ref, bbuv_ref,
    wq_ref, bq_ref, wk_ref, bk_ref, wv_ref, bv_ref,
    fc_w_ref, fc_b_ref, ln_g_ref, ln_b_ref,
    gbu_w_ref, gbu_b_ref,
    out_ref,
    qc_ref,
    wkc_ref,
    bkc_ref,
    wvc_ref,
    bvc_ref,
    m_ref,
    l_ref,
    acc_ref,
):
    i = pl.program_id(0)

    @pl.when(i == 0)
    def _init():
        q0 = _lin(h_tissue_ref[...], wbuq_ref[...], bbuq_ref[...])
        qc = _lin(q0, wq_ref[...], bq_ref[...])
        qc_ref[...] = qc * (1.0 / math.sqrt(DK))
        wkc_ref[...] = jnp.dot(wk_ref[...], wbuk_ref[...], precision=_PREC)
        bkc_ref[...] = _lin(bbuk_ref[...].reshape(1, D), wk_ref[...],
                            bk_ref[...])
        wvc_ref[...] = jnp.dot(wv_ref[...], wbuv_ref[...], precision=_PREC)
        bvc_ref[...] = _lin(bbuv_ref[...].reshape(1, D), wv_ref[...],
                            bv_ref[...])
        m_ref[...] = jnp.full((K, H), -1e30, jnp.float32)
        l_ref[...] = jnp.zeros((K, H), jnp.float32)
        acc_ref[...] = jnp.zeros((H, K, DK), jnp.float32)

    kc = _lin(h_cell_ref[...], wkc_ref[...], bkc_ref[...])
    vc = _lin(h_cell_ref[...], wvc_ref[...], bvc_ref[...])
    mask = st_ref[...] > 0.1

    for h in range(H):
        q_h = qc_ref[:, h * DK:(h + 1) * DK]
        k_h = kc[:, h * DK:(h + 1) * DK]
        v_h = vc[:, h * DK:(h + 1) * DK]
        s = jax.lax.dot_general(q_h, k_h, (((1,), (1,)), ((), ())),
                                precision=_PREC)
        s = jnp.where(mask, s, -jnp.inf)
        m_old = m_ref[:, h:h + 1]
        m_new = jnp.maximum(m_old, jnp.max(s, axis=1, keepdims=True))
        m_new = jnp.maximum(m_new, -1e30)
        alpha = jnp.exp(m_old - m_new)
        p = jnp.exp(s - m_new)
        l_ref[:, h:h + 1] = (l_ref[:, h:h + 1] * alpha
                             + jnp.sum(p, axis=1, keepdims=True))
        pv = jnp.dot(p, v_h, precision=_PREC)
        acc_ref[h, :, :] = acc_ref[h, :, :] * alpha + pv
        m_ref[:, h:h + 1] = m_new

    @pl.when(i == pl.num_programs(0) - 1)
    def _finish():
        parts = []
        for h in range(H):
            denom = jnp.maximum(l_ref[:, h:h + 1], 1e-30)
            parts.append(acc_ref[h, :, :] / denom)
        attn = jnp.concatenate(parts, axis=1)
        attn = _lin(attn, fc_w_ref[...], fc_b_ref[...])
        attn = _layer_norm(attn, ln_g_ref[...], ln_b_ref[...])
        ht = h_tissue_ref[...]
        gate = jax.nn.sigmoid(
            _lin(ht, gbu_w_ref[:, :D])
            + _lin(attn, gbu_w_ref[:, D:])
            + gbu_b_ref[...])
        new_row = gate * attn + (1.0 - gate) * ht
        has_any = l_ref[:, 0:1] > 0.0
        out_ref[...] = jnp.where(has_any, new_row, ht)


def _td_kernel(
    h_cell_ref,
    s_ref,
    ht_upd_ref,
    wtdv_ref, btdv_ref, wv_ref, bv_ref,
    fc_w_ref, fc_b_ref, ln_g_ref, ln_b_ref,
    gtd_w_ref, gtd_b_ref,
    out_ref,
    table_ref,
    gtab_ref,
):
    i = pl.program_id(0)

    @pl.when(i == 0)
    def _init():
        v0 = _lin(ht_upd_ref[...], wtdv_ref[...], btdv_ref[...])
        v1 = _lin(v0, wv_ref[...], bv_ref[...])
        t = _lin(v1, fc_w_ref[...], fc_b_ref[...])
        t = _layer_norm(t, ln_g_ref[...], ln_b_ref[...])
        table_ref[...] = t
        gtab_ref[...] = _lin(t, gtd_w_ref[:, D:])

    s = s_ref[...]
    rowmax = jnp.max(s, axis=1, keepdims=True)
    eq = s == rowmax
    col = jax.lax.broadcasted_iota(jnp.int32, (BB, K), 1)
    first = jnp.min(jnp.where(eq, col, K), axis=1, keepdims=True)
    onehot = (col == first).astype(jnp.float32)

    attn = jnp.dot(onehot, table_ref[...], precision=_PREC)
    g2 = jnp.dot(onehot, gtab_ref[...], precision=_PREC)
    hc = h_cell_ref[...]
    gate = jax.nn.sigmoid(_lin(hc, gtd_w_ref[:, :D]) + g2 + gtd_b_ref[...])
    out_ref[...] = gate * attn + (1.0 - gate) * hc


def _full(shape):
    return pl.BlockSpec(shape, lambda i: tuple(0 for _ in shape))


@jax.jit
def kernel(h_cell, h_tissue, S, params):
    p = params
    bu = p['bu']
    td = p['td']
    st = S.T

    w_full = _full((D, D))
    b_full = _full((D,))

    h_tissue_updated = pl.pallas_call(
        _bu_kernel,
        grid=(NBLK_A,),
        in_specs=[
            pl.BlockSpec((BA, D), lambda i: (i, 0)),
            pl.BlockSpec((K, BA), lambda i: (0, i)),
            _full((K, D)),
            w_full, b_full, w_full, b_full, w_full, b_full,
            w_full, b_full, w_full, b_full, w_full, b_full,
            w_full, b_full, b_full, b_full,
            _full((D, 2 * D)), b_full,
        ],
        out_specs=_full((K, D)),
        out_shape=jax.ShapeDtypeStruct((K, D), jnp.float32),
        scratch_shapes=[
            pltpu.VMEM((K, D), jnp.float32),
            pltpu.VMEM((D, D), jnp.float32),
            pltpu.VMEM((1, D), jnp.float32),
            pltpu.VMEM((D, D), jnp.float32),
            pltpu.VMEM((1, D), jnp.float32),
            pltpu.VMEM((K, H), jnp.float32),
            pltpu.VMEM((K, H), jnp.float32),
            pltpu.VMEM((H, K, DK), jnp.float32),
        ],
    )(
        h_cell, st, h_tissue,
        p['Wbuq_w'], p['Wbuq_b'], p['Wbuk_w'], p['Wbuk_b'],
        p['Wbuv_w'], p['Wbuv_b'],
        bu['Wq_w'], bu['Wq_b'], bu['Wk_w'], bu['Wk_b'],
        bu['Wv_w'], bu['Wv_b'],
        bu['fc_w'], bu['fc_b'], bu['ln_g'], bu['ln_b'],
        p['gbu_w'], p['gbu_b'],
    )

    h_cell_updated = pl.pallas_call(
        _td_kernel,
        grid=(NBLK_B,),
        in_specs=[
            pl.BlockSpec((BB, D), lambda i: (i, 0)),
            pl.BlockSpec((BB, K), lambda i: (i, 0)),
            _full((K, D)),
            w_full, b_full, w_full, b_full,
            w_full, b_full, b_full, b_full,
            _full((D, 2 * D)), b_full,
        ],
        out_specs=pl.BlockSpec((BB, D), lambda i: (i, 0)),
        out_shape=jax.ShapeDtypeStruct((N, D), jnp.float32),
        scratch_shapes=[
            pltpu.VMEM((K, D), jnp.float32),
            pltpu.VMEM((K, D), jnp.float32),
        ],
    )(
        h_cell, S, h_tissue_updated,
        p['Wtdv_w'], p['Wtdv_b'], td['Wv_w'], td['Wv_b'],
        td['fc_w'], td['fc_b'], td['ln_g'], td['ln_b'],
        p['gtd_w'], p['gtd_b'],
    )

    return h_cell_updated, h_tissue_updated


# speedup vs baseline: 3.4545x; 1.0216x over previous
import math

import jax
import jax.numpy as jnp
from jax.experimental import pallas as pl
from jax.experimental.pallas import tpu as pltpu

D = 256
H = 4
DK = D // H
N = 4096
K = 16

_PREC = jax.lax.Precision.HIGHEST


def _lin(x, w, b=None):
    out = jax.lax.dot_general(x, w, (((1,), (1,)), ((), ())), precision=_PREC)
    if b is not None:
        out = out + b
    return out


def _layer_norm(x, g, b, eps=1e-5):
    mu = jnp.mean(x, axis=-1, keepdims=True)
    xc = x - mu
    var = jnp.mean(xc * xc, axis=-1, keepdims=True)
    return xc * jax.lax.rsqrt(var + eps) * g + b


def _fused_kernel(
    h_cell_ref,
    s_ref,
    h_tissue_ref,
    wbuq_ref, bbuq_ref, wbuk_ref, bbuk_ref, wbuv_ref, bbuv_ref,
    buq_ref, bubq_ref, buk_ref, bubk_ref, buv_ref, bubv_ref,
    bufc_w_ref, bufc_b_ref, buln_g_ref, buln_b_ref,
    gbu_w_ref, gbu_b_ref,
    wtdv_ref, btdv_ref, tdv_ref, tdbv_ref,
    tdfc_w_ref, tdfc_b_ref, tdln_g_ref, tdln_b_ref,
    gtd_w_ref, gtd_b_ref,
    out_cell_ref,
    out_tissue_ref,
):
    hc = h_cell_ref[...]
    ht = h_tissue_ref[...]
    s_raw = s_ref[...]

    q0 = _lin(ht, wbuq_ref[...], bbuq_ref[...])
    qc = _lin(q0, buq_ref[...], bubq_ref[...]) * (1.0 / math.sqrt(DK))
    wkc = jnp.dot(buk_ref[...], wbuk_ref[...], precision=_PREC)
    bkc = _lin(bbuk_ref[...].reshape(1, D), buk_ref[...], bubk_ref[...])
    wvc = jnp.dot(buv_ref[...], wbuv_ref[...], precision=_PREC)
    bvc = _lin(bbuv_ref[...].reshape(1, D), buv_ref[...], bubv_ref[...])
    kc = _lin(hc, wkc, bkc)
    vc = _lin(hc, wvc, bvc)

    mask_t = jnp.transpose(s_raw) > 0.1
    parts = []
    l0 = None
    for h in range(H):
        q_h = qc[:, h * DK:(h + 1) * DK]
        k_h = kc[:, h * DK:(h + 1) * DK]
        v_h = vc[:, h * DK:(h + 1) * DK]
        s = jax.lax.dot_general(q_h, k_h, (((1,), (1,)), ((), ())),
                                precision=_PREC)
        s = jnp.where(mask_t, s, -jnp.inf)
        m = jnp.maximum(jnp.max(s, axis=1, keepdims=True), -1e30)
        p = jnp.exp(s - m)
        l = jnp.sum(p, axis=1, keepdims=True)
        if h == 0:
            l0 = l
        pv = jnp.dot(p, v_h, precision=_PREC)
        parts.append(pv / jnp.maximum(l, 1e-30))
    attn = jnp.concatenate(parts, axis=1)
    attn = _lin(attn, bufc_w_ref[...], bufc_b_ref[...])
    attn = _layer_norm(attn, buln_g_ref[...], buln_b_ref[...])
    gate = jax.nn.sigmoid(
        _lin(ht, gbu_w_ref[:, :D])
        + _lin(attn, gbu_w_ref[:, D:])
        + gbu_b_ref[...])
    new_rows = gate * attn + (1.0 - gate) * ht
    ht_upd = jnp.where(l0 > 0.0, new_rows, ht)
    out_tissue_ref[...] = ht_upd

    v0 = _lin(ht_upd, wtdv_ref[...], btdv_ref[...])
    v1 = _lin(v0, tdv_ref[...], tdbv_ref[...])
    table = _lin(v1, tdfc_w_ref[...], tdfc_b_ref[...])
    table = _layer_norm(table, tdln_g_ref[...], tdln_b_ref[...])
    gtab = _lin(table, gtd_w_ref[:, D:])

    rowmax = jnp.max(s_raw, axis=1, keepdims=True)
    eq = s_raw == rowmax
    col = jax.lax.broadcasted_iota(jnp.int32, (N, K), 1)
    first = jnp.min(jnp.where(eq, col, K), axis=1, keepdims=True)
    onehot = (col == first).astype(jnp.float32)

    attn_c = jnp.dot(onehot, table, precision=_PREC)
    g2 = jnp.dot(onehot, gtab, precision=_PREC)
    gate_c = jax.nn.sigmoid(_lin(hc, gtd_w_ref[:, :D]) + g2 + gtd_b_ref[...])
    out_cell_ref[...] = gate_c * attn_c + (1.0 - gate_c) * hc


@jax.jit
def kernel(h_cell, h_tissue, S, params):
    p = params
    bu = p['bu']
    td = p['td']

    out_cell, out_tissue = pl.pallas_call(
        _fused_kernel,
        out_shape=(
            jax.ShapeDtypeStruct((N, D), jnp.float32),
            jax.ShapeDtypeStruct((K, D), jnp.float32),
        ),
    )(
        h_cell, S, h_tissue,
        p['Wbuq_w'], p['Wbuq_b'], p['Wbuk_w'], p['Wbuk_b'],
        p['Wbuv_w'], p['Wbuv_b'],
        bu['Wq_w'], bu['Wq_b'], bu['Wk_w'], bu['Wk_b'],
        bu['Wv_w'], bu['Wv_b'],
        bu['fc_w'], bu['fc_b'], bu['ln_g'], bu['ln_b'],
        p['gbu_w'], p['gbu_b'],
        p['Wtdv_w'], p['Wtdv_b'], td['Wv_w'], td['Wv_b'],
        td['fc_w'], td['fc_b'], td['ln_g'], td['ln_b'],
        p['gtd_w'], p['gtd_b'],
    )
    return out_cell, out_tissue
